# Initial kernel scaffold; baseline (speedup 1.0000x reference)
import jax
import jax.numpy as jnp
from jax.experimental import pallas as pl


def kernel(x, edge_index, edge_weight, W1, att_src1, att_dst1, b1, bn_gamma, bn_beta, W2, att_src2, att_dst2, b2):
    raise NotImplementedError("write your pallas kernel here")


# speedup vs baseline: 55.0943x; 55.0943x over previous
import functools

import jax
import jax.numpy as jnp
from jax import lax
from jax.experimental import pallas as pl
from jax.experimental.pallas import tpu as pltpu
from jax.experimental.pallas import tpu_sc as plsc

G = 4
N = 10000
E = 320000
H = 8
DH = 8
F = H * DH
OUT = 4

GN = G * N
CPG = 2
NSUB = 16
EPP = E // NSUB
CH = 80
NCHUNK = EPP // CH
WB = 16
TW = 128
BLK = 40
NBLK = N // BLK
BPS = NBLK // NSUB
REM = NBLK - BPS * NSUB
BN = 1000

_mesh = plsc.VectorSubcoreMesh(core_axis_name="c", subcore_axis_name="s")


def _take(v, idx):
    return v.at[idx].get(mode="promise_in_bounds")


@functools.partial(
    pl.kernel, mesh=_mesh,
    out_type=jax.ShapeDtypeStruct((GN, TW), jnp.float32),
    scratch_types=[
        pltpu.VMEM((CH,), jnp.int32),
        pltpu.VMEM((CH,), jnp.int32),
        pltpu.VMEM((CH,), jnp.int32),
        pltpu.VMEM((CH,), jnp.float32),
        pltpu.VMEM((CH, TW), jnp.float32),
        pltpu.VMEM((CH, TW), jnp.float32),
        pltpu.VMEM((CH, TW), jnp.float32),
        pltpu.VMEM((BLK, TW), jnp.float32),
        pltpu.VMEM_SHARED((N, TW), jnp.float32),
        pltpu.SemaphoreType.DMA,
        pltpu.SemaphoreType.DMA,
    ])
def _sc_layer1(tab1, srcf, dstf, ewf, acc_out,
               srcv, dgv, dlv, ewv, srows, drows, mrows,
               zbuf, acc, sem1, sem2):
    c = lax.axis_index("c")
    s = lax.axis_index("s")
    zero16 = jnp.zeros((16,), jnp.float32)
    lo_idx = lax.iota(jnp.int32, 16) & 7
    hi_idx = lo_idx + 8

    @pl.loop(0, BLK)
    def _(i):
        for k in range(TW // 16):
            zbuf[i, pl.ds(16 * k, 16)] = zero16

    @pl.loop(0, CH)
    def _(e):
        for k in range(TW // 16):
            mrows[e, pl.ds(16 * k, 16)] = zero16

    for p in range(CPG):
        g = CPG * c + p
        gofs = g * N

        @pl.loop(0, BPS)
        def _(i):
            r0 = (s * BPS + i) * BLK
            pltpu.sync_copy(zbuf, acc.at[pl.ds(r0, BLK)])

        @pl.when(s < REM)
        def _():
            r0 = (NSUB * BPS + s) * BLK
            pltpu.sync_copy(zbuf, acc.at[pl.ds(r0, BLK)])

        plsc.subcore_barrier()

        ebase = g * E + s * EPP

        @pl.loop(0, NCHUNK)
        def _(i):
            off = ebase + i * CH
            pltpu.sync_copy(srcf.at[pl.ds(off, CH)], srcv)
            pltpu.sync_copy(dstf.at[pl.ds(off, CH)], dgv)
            pltpu.sync_copy(ewf.at[pl.ds(off, CH)], ewv)
            for k in range(CH // 16):
                dlv[pl.ds(16 * k, 16)] = dgv[pl.ds(16 * k, 16)] - gofs
            cp1 = pltpu.async_copy(tab1.at[srcv], srows, sem1)
            cp2 = pltpu.async_copy(tab1.at[dgv], drows, sem2)
            cp1.wait()
            cp2.wait()

            @pl.loop(0, CH, step=16)
            def _(e0):
                ew16 = ewv[pl.ds(e0, 16)]
                for j in range(16):
                    e = e0 + j
                    a4 = srows[e, pl.ds(F, 16)]
                    b = drows[e, pl.ds(F + 16, 16)]
                    u = a4 * b
                    v = _take(u, hi_idx)
                    ex = jnp.maximum(u, v)
                    ews = _take(ew16, jnp.full((16,), j, jnp.int32))
                    w2 = _take(ex * ews, lo_idx)
                    for k in range(4):
                        mrows[e, pl.ds(16 * k, 16)] = (
                            srows[e, pl.ds(16 * k, 16)] * w2)
                    mrows[e, pl.ds(F, 16)] = ex

            pltpu.sync_copy(mrows, acc.at[dlv], add=True)

        plsc.subcore_barrier()

        @pl.loop(0, BPS)
        def _(i):
            r0 = (s * BPS + i) * BLK
            pltpu.sync_copy(acc.at[pl.ds(r0, BLK)],
                            acc_out.at[pl.ds(gofs + r0, BLK)])

        @pl.when(s < REM)
        def _():
            r0 = (NSUB * BPS + s) * BLK
            pltpu.sync_copy(acc.at[pl.ds(r0, BLK)],
                            acc_out.at[pl.ds(gofs + r0, BLK)])

        if p + 1 < CPG:
            plsc.subcore_barrier()


@functools.partial(
    pl.kernel, mesh=_mesh,
    out_type=jax.ShapeDtypeStruct((GN, TW), jnp.float32),
    scratch_types=[
        pltpu.VMEM((CH,), jnp.int32),
        pltpu.VMEM((CH,), jnp.int32),
        pltpu.VMEM((CH,), jnp.int32),
        pltpu.VMEM((CH,), jnp.float32),
        pltpu.VMEM((CH, TW), jnp.float32),
        pltpu.VMEM((CH, TW), jnp.float32),
        pltpu.VMEM((CH, TW), jnp.float32),
        pltpu.VMEM((BLK, TW), jnp.float32),
        pltpu.VMEM_SHARED((N, TW), jnp.float32),
        pltpu.SemaphoreType.DMA,
        pltpu.SemaphoreType.DMA,
    ])
def _sc_layer2(tab2, srcf, dstf, ewf, acc_out,
               srcv, dgv, dlv, ewv, srows, drows, mrows,
               zbuf, acc, sem1, sem2):
    c = lax.axis_index("c")
    s = lax.axis_index("s")
    zero16 = jnp.zeros((16,), jnp.float32)
    lane = lax.iota(jnp.int32, 16)
    lt4 = lane < 4
    i4 = jnp.full((16,), 4, jnp.int32)
    i5 = jnp.full((16,), 5, jnp.int32)
    dmap = jnp.where(lane == 4, 6, jnp.where(lane == 5, 7, 0))

    @pl.loop(0, BLK)
    def _(i):
        for k in range(TW // 16):
            zbuf[i, pl.ds(16 * k, 16)] = zero16

    @pl.loop(0, CH)
    def _(e):
        for k in range(TW // 16):
            mrows[e, pl.ds(16 * k, 16)] = zero16

    for p in range(CPG):
        g = CPG * c + p
        gofs = g * N

        @pl.loop(0, BPS)
        def _(i):
            pltpu.sync_copy(zbuf, acc.at[pl.ds((s * BPS + i) * BLK, BLK)])

        @pl.when(s < REM)
        def _():
            pltpu.sync_copy(zbuf, acc.at[pl.ds((NSUB * BPS + s) * BLK, BLK)])

        plsc.subcore_barrier()

        ebase = g * E + s * EPP

        @pl.loop(0, NCHUNK)
        def _(i):
            off = ebase + i * CH
            pltpu.sync_copy(srcf.at[pl.ds(off, CH)], srcv)
            pltpu.sync_copy(dstf.at[pl.ds(off, CH)], dgv)
            pltpu.sync_copy(ewf.at[pl.ds(off, CH)], ewv)
            for k in range(CH // 16):
                dlv[pl.ds(16 * k, 16)] = dgv[pl.ds(16 * k, 16)] - gofs
            cp1 = pltpu.async_copy(tab2.at[srcv], srows, sem1)
            cp2 = pltpu.async_copy(tab2.at[dgv], drows, sem2)
            cp1.wait()
            cp2.wait()

            @pl.loop(0, CH, step=16)
            def _(e0):
                ew16 = ewv[pl.ds(e0, 16)]
                for j in range(16):
                    e = e0 + j
                    a = srows[e, pl.ds(0, 16)]
                    d = drows[e, pl.ds(0, 16)]
                    u = a * _take(d, dmap)
                    exb = jnp.maximum(_take(u, i4), _take(u, i5))
                    ews = _take(ew16, jnp.full((16,), j, jnp.int32))
                    m = a * (exb * ews)
                    mrows[e, pl.ds(0, 16)] = jnp.where(lt4, m, exb)

            pltpu.sync_copy(mrows, acc.at[dlv], add=True)

        plsc.subcore_barrier()

        @pl.loop(0, BPS)
        def _(i):
            r0 = (s * BPS + i) * BLK
            pltpu.sync_copy(acc.at[pl.ds(r0, BLK)],
                            acc_out.at[pl.ds(gofs + r0, BLK)])

        @pl.when(s < REM)
        def _():
            r0 = (NSUB * BPS + s) * BLK
            pltpu.sync_copy(acc.at[pl.ds(r0, BLK)],
                            acc_out.at[pl.ds(gofs + r0, BLK)])

        if p + 1 < CPG:
            plsc.subcore_barrier()


def _tc1_body(x_ref, w1_ref, asrc_ref, adst_ref, tab1_ref):
    h = jnp.dot(x_ref[0], w1_ref[0], preferred_element_type=jnp.float32)
    a_s = jnp.dot(h, asrc_ref[0], preferred_element_type=jnp.float32)
    a_d = jnp.dot(h, adst_ref[0], preferred_element_type=jnp.float32)
    z = jnp.zeros((h.shape[0], TW - F - 4 * H), jnp.float32)
    tab1_ref[0] = jnp.concatenate(
        [h, jnp.exp(a_s), jnp.exp(0.2 * a_s),
         jnp.exp(a_d), jnp.exp(0.2 * a_d), z], axis=1)


def _tc2_body(acc1_ref, b1_ref, gam_ref, bet_ref, w2_ref, as2_ref,
              ad2_ref, tab2_ref):
    msg = acc1_ref[0][:, 0:F]
    den = acc1_ref[0][:, F:F + 8]
    o = msg / (jnp.tile(den, (1, 8)) + 1e-16) + b1_ref[0, 0]
    o = jnp.where(o > 0, o, jnp.exp(jnp.minimum(o, 0.0)) - 1.0)
    hbn = o * (1.0 / jnp.sqrt(1.0 + 1e-5)) * gam_ref[0, 0] + bet_ref[0, 0]
    h2 = jnp.dot(hbn, w2_ref[...], preferred_element_type=jnp.float32)
    a_s2 = jnp.dot(h2, as2_ref[...], preferred_element_type=jnp.float32)
    a_d2 = jnp.dot(h2, ad2_ref[...], preferred_element_type=jnp.float32)
    z = jnp.zeros((h2.shape[0], TW - 8), jnp.float32)
    tab2_ref[0] = jnp.concatenate(
        [h2, jnp.exp(a_s2), jnp.exp(0.2 * a_s2),
         jnp.exp(a_d2), jnp.exp(0.2 * a_d2), z], axis=1)


def _tc3_body(acc2_ref, b2_ref, out_ref):
    a = acc2_ref[...]
    o = a[:, :, 0:4] / (a[:, :, 4:5] + 1e-16) + b2_ref[...]
    m = jnp.max(o, axis=-1, keepdims=True)
    z = o - m
    lse = jnp.log(jnp.sum(jnp.exp(z), axis=-1, keepdims=True))
    out_ref[...] = (z - lse).transpose(1, 0, 2)


def kernel(x, edge_index, edge_weight, W1, att_src1, att_dst1, b1,
           bn_gamma, bn_beta, W2, att_src2, att_dst2, b2):
    f32 = jnp.float32
    cols = jnp.arange(F)
    perm = (cols % H) * DH + cols // H
    W1p = W1[:, :, perm]
    eye_rep = jnp.tile(jnp.eye(H, dtype=f32), (DH, 1))
    asrc_p = att_src1.transpose(0, 2, 1).reshape(G, F, 1) * eye_rep[None]
    adst_p = att_dst1.transpose(0, 2, 1).reshape(G, F, 1) * eye_rep[None]
    b1p = b1[:, perm]
    gamp = bn_gamma[:, perm]
    betp = bn_beta[:, perm]
    W2p = W2[perm, :]
    as2c = att_src2.reshape(OUT, 1)
    ad2c = att_dst2.reshape(OUT, 1)

    goff = (jnp.arange(G, dtype=jnp.int32) * N)[:, None]
    srcf = (edge_index[:, 0, :] + goff).reshape(-1)
    dstf = (edge_index[:, 1, :] + goff).reshape(-1)
    ewf = edge_weight.reshape(-1)

    nb = N // BN
    tab1 = pl.pallas_call(
        _tc1_body,
        grid=(G, nb),
        in_specs=[
            pl.BlockSpec((1, BN, 2), lambda g, i: (g, i, 0)),
            pl.BlockSpec((1, 2, F), lambda g, i: (g, 0, 0)),
            pl.BlockSpec((1, F, H), lambda g, i: (g, 0, 0)),
            pl.BlockSpec((1, F, H), lambda g, i: (g, 0, 0)),
        ],
        out_specs=pl.BlockSpec((1, BN, TW), lambda g, i: (g, i, 0)),
        out_shape=jax.ShapeDtypeStruct((G, N, TW), f32),
    )(x, W1p, asrc_p, adst_p)

    acc1 = _sc_layer1(tab1.reshape(GN, TW), srcf, dstf, ewf)

    tab2 = pl.pallas_call(
        _tc2_body,
        grid=(G, nb),
        in_specs=[
            pl.BlockSpec((1, BN, TW), lambda g, i: (g, i, 0)),
            pl.BlockSpec((1, 1, F), lambda g, i: (g, 0, 0)),
            pl.BlockSpec((1, 1, F), lambda g, i: (g, 0, 0)),
            pl.BlockSpec((1, 1, F), lambda g, i: (g, 0, 0)),
            pl.BlockSpec((F, OUT), lambda g, i: (0, 0)),
            pl.BlockSpec((OUT, 1), lambda g, i: (0, 0)),
            pl.BlockSpec((OUT, 1), lambda g, i: (0, 0)),
        ],
        out_specs=pl.BlockSpec((1, BN, TW), lambda g, i: (g, i, 0)),
        out_shape=jax.ShapeDtypeStruct((G, N, TW), f32),
    )(acc1.reshape(G, N, TW), b1p.reshape(G, 1, F),
      gamp.reshape(G, 1, F), betp.reshape(G, 1, F), W2p, as2c, ad2c)

    acc2 = _sc_layer2(tab2.reshape(GN, TW), srcf, dstf, ewf)

    out = pl.pallas_call(
        _tc3_body,
        grid=(nb,),
        in_specs=[
            pl.BlockSpec((G, BN, TW), lambda i: (0, i, 0)),
            pl.BlockSpec((1, 1, OUT), lambda i: (0, 0, 0)),
        ],
        out_specs=pl.BlockSpec((BN, G, OUT), lambda i: (i, 0, 0)),
        out_shape=jax.ShapeDtypeStruct((N, G, OUT), f32),
    )(acc2.reshape(G, N, TW), b2.reshape(1, 1, OUT))

    return out
